# baseline (speedup 1.0000x reference)
import functools

import jax
import jax.numpy as jnp
from jax import lax
from jax.experimental import pallas as pl
from jax.experimental.pallas import tpu as pltpu
from jax.experimental.pallas import tpu_sc as plsc

EMBED_DIMENSION = 300
EMBED_MAX_NORM = 1.0
VOCAB = 100000
BATCH = 1024

N_TILE = 4096

_NC = 2
_NS = 16
_NW = _NC * _NS
_B_PER_W = BATCH // _NW


def _sc_pair_gather(inputs, emb_table):
    table2 = emb_table.reshape(VOCAB // 2, 2 * EMBED_DIMENSION)
    mesh = plsc.VectorSubcoreMesh(core_axis_name="c", subcore_axis_name="s")

    @functools.partial(
        pl.kernel,
        mesh=mesh,
        compiler_params=pltpu.CompilerParams(use_tc_tiling_on_sc=False),
        out_type=jax.ShapeDtypeStruct((BATCH, 2 * EMBED_DIMENSION), jnp.float32),
        scratch_types=[
            pltpu.VMEM((_B_PER_W,), jnp.int32),
            pltpu.VMEM((_B_PER_W,), jnp.int32),
            pltpu.VMEM((_B_PER_W, 2 * EMBED_DIMENSION), jnp.float32),
            pltpu.SemaphoreType.DMA,
        ],
    )
    def k(idx_hbm, table_hbm, out_hbm, idx_v, half_v, rows_v, sem):
        wid = lax.axis_index("s") * _NC + lax.axis_index("c")
        base = wid * _B_PER_W
        pltpu.sync_copy(idx_hbm.at[pl.ds(base, _B_PER_W)], idx_v)
        for c in range(_B_PER_W // 16):
            v16 = idx_v[pl.ds(c * 16, 16)]
            half_v[pl.ds(c * 16, 16)] = lax.shift_right_logical(v16, 1)
        pltpu.async_copy(table_hbm.at[half_v], rows_v, sem).wait()
        pltpu.sync_copy(rows_v, out_hbm.at[pl.ds(base, _B_PER_W)])

    return k(inputs, table2)


def _matmul_kernel(emb_ref, w_ref, b_ref, out_ref, ebf_ref):
    @pl.when(pl.program_id(0) == 0)
    def _():
        e = emb_ref[...]
        nrm = jnp.sqrt(jnp.sum(e * e, axis=1, keepdims=True))
        scale = jnp.minimum(1.0, EMBED_MAX_NORM / jnp.maximum(nrm, 1e-7))
        ebf_ref[...] = (e * scale).astype(jnp.bfloat16)

    e = ebf_ref[...]
    w = w_ref[...].astype(jnp.bfloat16)
    acc = jax.lax.dot_general(
        e, w, (((1,), (1,)), ((), ())), preferred_element_type=jnp.float32
    )
    out_ref[...] = acc + b_ref[0, :][None, :]


def _projection(emb, W, b):
    n_blocks = pl.cdiv(VOCAB, N_TILE)
    b2 = b.reshape(1, VOCAB)
    return pl.pallas_call(
        _matmul_kernel,
        grid=(n_blocks,),
        in_specs=[
            pl.BlockSpec((BATCH, EMBED_DIMENSION), lambda j: (0, 0)),
            pl.BlockSpec((N_TILE, EMBED_DIMENSION), lambda j: (j, 0)),
            pl.BlockSpec((1, N_TILE), lambda j: (0, j)),
        ],
        out_specs=pl.BlockSpec((BATCH, N_TILE), lambda j: (0, j)),
        out_shape=jax.ShapeDtypeStruct((BATCH, VOCAB), jnp.float32),
        scratch_shapes=[pltpu.VMEM((BATCH, EMBED_DIMENSION), jnp.bfloat16)],
    )(emb, W, b2)


@jax.jit
def kernel(inputs, emb_table, W, b):
    pairs = _sc_pair_gather(inputs, emb_table)
    odd = (inputs % 2 == 1)[:, None]
    emb = jnp.where(odd, pairs[:, EMBED_DIMENSION:], pairs[:, :EMBED_DIMENSION])
    return _projection(emb, W, b)


# speedup vs baseline: 1.6908x; 1.6908x over previous
import functools

import jax
import jax.numpy as jnp
from jax import lax
from jax.experimental import pallas as pl
from jax.experimental.pallas import tpu as pltpu

EMBED_DIMENSION = 300
EMBED_MAX_NORM = 1.0
VOCAB = 100000
BATCH = 1024

N_TILE = 4096


def _fused_kernel(idx_ref, table_ref, w_ref, b_ref, out_ref, raw_ref, ebf_ref, sem):
    @pl.when(pl.program_id(0) == 0)
    def _():
        def issue(r, carry):
            pltpu.make_async_copy(
                table_ref.at[pl.ds(idx_ref[r], 1), :],
                raw_ref.at[pl.ds(r, 1), :],
                sem,
            ).start()
            return carry

        lax.fori_loop(0, BATCH, issue, 0)

        def drain(r, carry):
            pltpu.make_async_copy(
                table_ref.at[pl.ds(0, 1), :],
                raw_ref.at[pl.ds(0, 1), :],
                sem,
            ).wait()
            return carry

        lax.fori_loop(0, BATCH, drain, 0)

        e = raw_ref[...]
        nrm = jnp.sqrt(jnp.sum(e * e, axis=1, keepdims=True))
        scale = jnp.minimum(1.0, EMBED_MAX_NORM / jnp.maximum(nrm, 1e-7))
        ebf_ref[...] = (e * scale).astype(jnp.bfloat16)

    e = ebf_ref[...]
    w = w_ref[...].astype(jnp.bfloat16)
    acc = jax.lax.dot_general(
        e, w, (((1,), (1,)), ((), ())), preferred_element_type=jnp.float32
    )
    out_ref[...] = acc + b_ref[0, :][None, :]


@jax.jit
def kernel(inputs, emb_table, W, b):
    n_blocks = pl.cdiv(VOCAB, N_TILE)
    b2 = b.reshape(1, VOCAB)
    return pl.pallas_call(
        _fused_kernel,
        grid_spec=pltpu.PrefetchScalarGridSpec(
            num_scalar_prefetch=1,
            grid=(n_blocks,),
            in_specs=[
                pl.BlockSpec(memory_space=pl.ANY),
                pl.BlockSpec((N_TILE, EMBED_DIMENSION), lambda j, idx: (j, 0)),
                pl.BlockSpec((1, N_TILE), lambda j, idx: (0, j)),
            ],
            out_specs=pl.BlockSpec((BATCH, N_TILE), lambda j, idx: (0, j)),
            scratch_shapes=[
                pltpu.VMEM((BATCH, EMBED_DIMENSION), jnp.float32),
                pltpu.VMEM((BATCH, EMBED_DIMENSION), jnp.bfloat16),
                pltpu.SemaphoreType.DMA,
            ],
        ),
        out_shape=jax.ShapeDtypeStruct((BATCH, VOCAB), jnp.float32),
    )(inputs, emb_table, W, b2)


# speedup vs baseline: 1.7097x; 1.0112x over previous
import functools

import jax
import jax.numpy as jnp
from jax import lax
from jax.experimental import pallas as pl
from jax.experimental.pallas import tpu as pltpu

EMBED_DIMENSION = 300
EMBED_MAX_NORM = 1.0
VOCAB = 100000
BATCH = 1024

N_TILE = 4096


def _fused_kernel(idx_ref, table_ref, w_ref, b_ref, out_ref, raw_ref, ebf_ref, sem):
    @pl.when(pl.program_id(0) == 0)
    def _():
        def issue(g, carry):
            for u in range(8):
                r = g * 8 + u
                pltpu.make_async_copy(
                    table_ref.at[pl.ds(idx_ref[r], 1), :],
                    raw_ref.at[pl.ds(r, 1), :],
                    sem,
                ).start()
            return carry

        lax.fori_loop(0, BATCH // 8, issue, 0)

        pltpu.make_async_copy(
            table_ref.at[pl.ds(0, BATCH), :], raw_ref, sem
        ).wait()

        e = raw_ref[...]
        nrm = jnp.sqrt(jnp.sum(e * e, axis=1, keepdims=True))
        scale = jnp.minimum(1.0, EMBED_MAX_NORM / jnp.maximum(nrm, 1e-7))
        ebf_ref[...] = (e * scale).astype(jnp.bfloat16)

    e = ebf_ref[...]
    w = w_ref[...].astype(jnp.bfloat16)
    acc = jax.lax.dot_general(
        e, w, (((1,), (1,)), ((), ())), preferred_element_type=jnp.float32
    )
    out_ref[...] = acc + b_ref[0, :][None, :]


@jax.jit
def kernel(inputs, emb_table, W, b):
    n_blocks = pl.cdiv(VOCAB, N_TILE)
    b2 = b.reshape(1, VOCAB)
    return pl.pallas_call(
        _fused_kernel,
        grid_spec=pltpu.PrefetchScalarGridSpec(
            num_scalar_prefetch=1,
            grid=(n_blocks,),
            in_specs=[
                pl.BlockSpec(memory_space=pl.ANY),
                pl.BlockSpec((N_TILE, EMBED_DIMENSION), lambda j, idx: (j, 0)),
                pl.BlockSpec((1, N_TILE), lambda j, idx: (0, j)),
            ],
            out_specs=pl.BlockSpec((BATCH, N_TILE), lambda j, idx: (0, j)),
            scratch_shapes=[
                pltpu.VMEM((BATCH, EMBED_DIMENSION), jnp.float32),
                pltpu.VMEM((BATCH, EMBED_DIMENSION), jnp.bfloat16),
                pltpu.SemaphoreType.DMA,
            ],
        ),
        out_shape=jax.ShapeDtypeStruct((BATCH, VOCAB), jnp.float32),
    )(inputs, emb_table, W, b2)
